# baseline (speedup 1.0000x reference)
import functools

import jax
import jax.numpy as jnp
from jax.experimental import pallas as pl
from jax.experimental.pallas import tpu as pltpu


def _dot(a, b):
    return jax.lax.dot_general(
        a, b, (((1,), (0,)), ((), ())), preferred_element_type=jnp.float32)


def _fused_kernel(x_ref, wr_ref, br_ref, wc_ref, bc_ref, w1_ref, b1_ref,
                  w5_ref, b5_ref, w6_ref, b6_ref, w7_ref, b7_ref,
                  o_ref, *, bB, naf):
    cols = []
    phs = []
    for g in range(bB):
        xg = x_ref[g]
        h = jnp.maximum(_dot(wr_ref[...], xg) + br_ref[...], 0.0)
        cols.append(jnp.sum(h, axis=1, keepdims=True))
        phs.append(xg[naf:naf + 3, 0:1])
    gT = jnp.concatenate(cols, axis=1)
    phT = jnp.concatenate(phs, axis=1)

    c = jnp.maximum(_dot(wc_ref[...], gT) + bc_ref[...], 0.0)
    d = jnp.maximum(_dot(w1_ref[...], c) + b1_ref[...], 0.0)
    d = _dot(w5_ref[...], d) + b5_ref[...]
    mv = _dot(w6_ref[...], d) + b6_ref[...]
    merged = jnp.concatenate([mv, phT], axis=0)
    o_ref[0] = _dot(w7_ref[...], merged) + b7_ref[...]


def kernel(inputs, W_rule, b_rule, W_conv, b_conv, W1, b1, W5, b5, W6, b6,
           W7, b7):
    B, N, F = inputs.shape
    naf, rule_out = W_rule.shape

    xT = jnp.transpose(inputs, (0, 2, 1))

    WrT = jnp.concatenate(
        [W_rule.T, jnp.zeros((rule_out, F - naf), W_rule.dtype)], axis=1)
    col = lambda v: v.reshape(-1, 1)

    bB = 16
    grid = (B // bB,)
    full = lambda a: pl.BlockSpec(a.shape, lambda b: (0,) * a.ndim)

    out = pl.pallas_call(
        functools.partial(_fused_kernel, bB=bB, naf=naf),
        grid=grid,
        in_specs=[
            pl.BlockSpec((bB, F, N), lambda b: (b, 0, 0)),
            full(WrT), full(col(b_rule)),
            full(W_conv.T), full(col(b_conv)),
            full(W1.T), full(col(b1)),
            full(W5.T), full(col(b5)),
            full(W6.T), full(col(b6)),
            full(W7.T), full(col(b7)),
        ],
        out_specs=pl.BlockSpec((1, 1, bB), lambda b: (b, 0, 0)),
        out_shape=jax.ShapeDtypeStruct((B // bB, 1, bB), jnp.float32),
        compiler_params=pltpu.CompilerParams(
            dimension_semantics=("arbitrary",)),
    )(xT, WrT, col(b_rule), W_conv.T, col(b_conv), W1.T, col(b1),
      W5.T, col(b5), W6.T, col(b6), W7.T, col(b7))
    return out.reshape(B, 1)


# speedup vs baseline: 1.3662x; 1.3662x over previous
import functools

import jax
import jax.numpy as jnp
from jax.experimental import pallas as pl
from jax.experimental.pallas import tpu as pltpu


def _dot(a, b):
    return jax.lax.dot_general(
        a, b, (((1,), (0,)), ((), ())), preferred_element_type=jnp.float32)


def _fused_kernel(x_ref, wr_ref, br_ref, wc_ref, bc_ref, w1_ref, b1_ref,
                  w5_ref, b5_ref, w6_ref, b6_ref, w7_ref, b7_ref,
                  o_ref, *, bB, naf):
    cols = []
    phs = []
    for g in range(bB):
        xg = x_ref[g]
        h = jnp.maximum(_dot(wr_ref[...], xg) + br_ref[...], 0.0)
        cols.append(jnp.sum(h, axis=1, keepdims=True))
        phs.append(xg[naf:naf + 3, 0:1])
    gT = jnp.concatenate(cols, axis=1)
    phT = jnp.concatenate(phs, axis=1)

    c = jnp.maximum(_dot(wc_ref[...], gT) + bc_ref[...], 0.0)
    d = jnp.maximum(_dot(w1_ref[...], c) + b1_ref[...], 0.0)
    d = _dot(w5_ref[...], d) + b5_ref[...]
    mv = _dot(w6_ref[...], d) + b6_ref[...]
    merged = jnp.concatenate([mv, phT], axis=0)
    o_ref[0] = _dot(w7_ref[...], merged) + b7_ref[...]


def kernel(inputs, W_rule, b_rule, W_conv, b_conv, W1, b1, W5, b5, W6, b6,
           W7, b7):
    B, N, F = inputs.shape
    naf, rule_out = W_rule.shape

    xT = jnp.transpose(inputs, (0, 2, 1))

    WrT = jnp.concatenate(
        [W_rule.T, jnp.zeros((rule_out, F - naf), W_rule.dtype)], axis=1)
    col = lambda v: v.reshape(-1, 1)

    bB = 32
    grid = (B // bB,)
    full = lambda a: pl.BlockSpec(a.shape, lambda b: (0,) * a.ndim)

    out = pl.pallas_call(
        functools.partial(_fused_kernel, bB=bB, naf=naf),
        grid=grid,
        in_specs=[
            pl.BlockSpec((bB, F, N), lambda b: (b, 0, 0)),
            full(WrT), full(col(b_rule)),
            full(W_conv.T), full(col(b_conv)),
            full(W1.T), full(col(b1)),
            full(W5.T), full(col(b5)),
            full(W6.T), full(col(b6)),
            full(W7.T), full(col(b7)),
        ],
        out_specs=pl.BlockSpec((1, 1, bB), lambda b: (b, 0, 0)),
        out_shape=jax.ShapeDtypeStruct((B // bB, 1, bB), jnp.float32),
        compiler_params=pltpu.CompilerParams(
            dimension_semantics=("arbitrary",)),
    )(xT, WrT, col(b_rule), W_conv.T, col(b_conv), W1.T, col(b1),
      W5.T, col(b5), W6.T, col(b6), W7.T, col(b7))
    return out.reshape(B, 1)


# speedup vs baseline: 1.6213x; 1.1867x over previous
import functools

import jax
import jax.numpy as jnp
from jax.experimental import pallas as pl
from jax.experimental.pallas import tpu as pltpu


def _dot(a, b):
    return jax.lax.dot_general(
        a, b, (((1,), (0,)), ((), ())), preferred_element_type=jnp.float32)


def _fused_kernel(x_ref, wr_ref, br_ref, wc_ref, bc_ref, w1_ref, b1_ref,
                  w5_ref, b5_ref, w6_ref, b6_ref, w7_ref, b7_ref,
                  o_ref, *, bB, naf):
    cols = []
    phs = []
    for g in range(bB):
        xg = x_ref[g]
        h = jnp.maximum(_dot(wr_ref[...], xg) + br_ref[...], 0.0)
        cols.append(jnp.sum(h, axis=1, keepdims=True))
        phs.append(xg[naf:naf + 3, 0:1])
    gT = jnp.concatenate(cols, axis=1)
    phT = jnp.concatenate(phs, axis=1)

    c = jnp.maximum(_dot(wc_ref[...], gT) + bc_ref[...], 0.0)
    d = jnp.maximum(_dot(w1_ref[...], c) + b1_ref[...], 0.0)
    d = _dot(w5_ref[...], d) + b5_ref[...]
    mv = _dot(w6_ref[...], d) + b6_ref[...]
    merged = jnp.concatenate([mv, phT], axis=0)
    o_ref[0] = _dot(w7_ref[...], merged) + b7_ref[...]


def kernel(inputs, W_rule, b_rule, W_conv, b_conv, W1, b1, W5, b5, W6, b6,
           W7, b7):
    B, N, F = inputs.shape
    naf, rule_out = W_rule.shape

    xT = jnp.transpose(inputs, (0, 2, 1))

    WrT = jnp.concatenate(
        [W_rule.T, jnp.zeros((rule_out, F - naf), W_rule.dtype)], axis=1)
    col = lambda v: v.reshape(-1, 1)

    bB = 64
    grid = (B // bB,)
    full = lambda a: pl.BlockSpec(a.shape, lambda b: (0,) * a.ndim)

    out = pl.pallas_call(
        functools.partial(_fused_kernel, bB=bB, naf=naf),
        grid=grid,
        in_specs=[
            pl.BlockSpec((bB, F, N), lambda b: (b, 0, 0)),
            full(WrT), full(col(b_rule)),
            full(W_conv.T), full(col(b_conv)),
            full(W1.T), full(col(b1)),
            full(W5.T), full(col(b5)),
            full(W6.T), full(col(b6)),
            full(W7.T), full(col(b7)),
        ],
        out_specs=pl.BlockSpec((1, 1, bB), lambda b: (b, 0, 0)),
        out_shape=jax.ShapeDtypeStruct((B // bB, 1, bB), jnp.float32),
        compiler_params=pltpu.CompilerParams(
            dimension_semantics=("arbitrary",)),
    )(xT, WrT, col(b_rule), W_conv.T, col(b_conv), W1.T, col(b1),
      W5.T, col(b5), W6.T, col(b6), W7.T, col(b7))
    return out.reshape(B, 1)
